# baseline (speedup 1.0000x reference)
import functools

import jax
import jax.numpy as jnp
from jax import lax
from jax.experimental import pallas as pl
from jax.experimental.pallas import tpu as pltpu
from jax.experimental.pallas import tpu_sc as plsc

NC = 2
NS = 16
NW = NC * NS
CH = 125


def _sc_gather(src2, xt):
    nw, ew = src2.shape
    f, n = xt.shape
    ewp = ((ew + 15) // 16) * 16
    ngr = ewp // 16
    mesh = plsc.VectorSubcoreMesh(core_axis_name="c", subcore_axis_name="s")

    @functools.partial(
        pl.kernel,
        out_type=jax.ShapeDtypeStruct((f, nw * ew), jnp.float32),
        mesh=mesh,
        compiler_params=pltpu.CompilerParams(use_tc_tiling_on_sc=False, needs_layout_passes=False),
        scratch_types=[
            pltpu.VMEM((f, n), jnp.float32),
            pltpu.VMEM((ewp,), jnp.int32),
            pltpu.VMEM((f, ewp), jnp.float32),
        ],
    )
    def gather_kernel(src_hbm, xt_hbm, xst_hbm, xt_v, idx_v, xst_v):
        wid = lax.axis_index("s") * NC + lax.axis_index("c")
        pltpu.sync_copy(xt_hbm, xt_v)
        pltpu.sync_copy(src_hbm.at[wid], idx_v.at[pl.ds(0, ew)])
        lanes = lax.iota(jnp.int32, 16)
        tail = idx_v[pl.ds(ewp - 16, 16)]
        idx_v[pl.ds(ewp - 16, 16)] = jnp.where(lanes < 16 - (ewp - ew),
                                               tail, 0)

        def body(q, carry):
            base = q * 16
            idx16 = idx_v[pl.ds(base, 16)]
            for ff in range(f):
                row = jnp.full((16,), ff, jnp.int32)
                xst_v[ff, pl.ds(base, 16)] = plsc.load_gather(
                    xt_v, [row, idx16])
            return carry

        lax.fori_loop(0, ngr, body, 0)
        pltpu.sync_copy(xst_v.at[:, pl.ds(0, ew)],
                        xst_hbm.at[:, pl.ds(wid * ew, ew)])

    return gather_kernel(src2, xt)


def _sc_scatter(dst3, msgt, n):
    nw, nch, ch = dst3.shape
    ew = nch * ch
    hch = 8
    nh = nch // hch
    hew = hch * ch
    unr = 5
    rpt = n // NS
    mesh = plsc.VectorSubcoreMesh(core_axis_name="c", subcore_axis_name="s")

    @functools.partial(
        pl.kernel,
        out_type=jax.ShapeDtypeStruct((NC, n, 16), jnp.float32),
        mesh=mesh,
        compiler_params=pltpu.CompilerParams(use_tc_tiling_on_sc=False, needs_layout_passes=False),
        scratch_types=[
            pltpu.VMEM((nch, ch), jnp.int32),
            pltpu.VMEM((16, hew + 1), jnp.float32),
            pltpu.VMEM((hch, ch, 16), jnp.float32),
            pltpu.VMEM((rpt, 16), jnp.float32),
            pltpu.VMEM_SHARED((n, 16), jnp.float32),
            pltpu.SemaphoreType.DMA,
        ],
    )
    def scatter_kernel(dst_hbm, msgt_hbm, out_hbm, idx_v, msgt_v, msg_v,
                       zero_v, agg_sh, sem):
        cid = lax.axis_index("c")
        sid = lax.axis_index("s")
        wid = sid * NC + cid
        lanes = lax.iota(jnp.int32, 16)

        def zbody(i, carry):
            zero_v[i] = jnp.zeros((16,), jnp.float32)
            return carry

        lax.fori_loop(0, rpt, zbody, 0)
        pltpu.sync_copy(zero_v, agg_sh.at[pl.ds(sid * rpt, rpt)])
        pltpu.sync_copy(dst_hbm.at[wid], idx_v)
        plsc.subcore_barrier()

        for h in range(nh):
            pltpu.sync_copy(
                msgt_hbm.at[:, pl.ds(wid * ew + h * hew, hew)],
                msgt_v.at[:, pl.ds(0, hew)])

            def tbody(t, carry):
                j = t // (ch // unr)
                p0 = (t % (ch // unr)) * unr
                for u in range(unr):
                    p = p0 + u
                    e = j * ch + p
                    v = plsc.load_gather(
                        msgt_v, [lanes, jnp.full((16,), e, jnp.int32)])
                    plsc.store_scatter(
                        msg_v, [jnp.full((16,), j, jnp.int32),
                                jnp.full((16,), p, jnp.int32), lanes], v)
                return carry

            lax.fori_loop(0, hch * (ch // unr), tbody, 0)

            def sbody(j, carry):
                pltpu.async_copy(msg_v.at[j],
                                 agg_sh.at[idx_v.at[h * hch + j]],
                                 sem, add=True)
                return carry

            lax.fori_loop(0, hch, sbody, 0)

            def dbody(j, carry):
                pltpu.make_async_copy(msg_v.at[j],
                                      agg_sh.at[idx_v.at[h * hch + j]],
                                      sem).wait()
                return carry

            lax.fori_loop(0, hch, dbody, 0)

        plsc.subcore_barrier()
        pltpu.sync_copy(agg_sh.at[pl.ds(sid * rpt, rpt)],
                        out_hbm.at[cid, pl.ds(sid * rpt, rpt)])

    return scatter_kernel(dst3, msgt)


def _tc_dense(pseudo_t, xs_t, w, cvec, rt):
    d, e = pseudo_t.shape
    koi = w.shape[0]
    o, oi = rt.shape
    k = koi // oi
    eb = 3200
    grid = e // eb

    def body(p_ref, xs_ref, w_ref, c_ref, rt_ref, out_ref):
        p = p_ref[...]
        fmat = jnp.concatenate([p * p, p], axis=0)
        arg = lax.dot_general(w_ref[...], fmat,
                              (((1,), (0,)), ((), ())),
                              preferred_element_type=jnp.float32)
        g = jnp.exp(-(arg + c_ref[...]))
        gm = g[0:oi]
        for kk in range(1, k):
            gm = gm + g[kk * oi:(kk + 1) * oi]
        prod = gm * jnp.tile(xs_ref[...], (o, 1))
        msg = lax.dot_general(rt_ref[...], prod,
                              (((1,), (0,)), ((), ())),
                              preferred_element_type=jnp.float32)
        out_ref[...] = jnp.concatenate(
            [msg,
             jnp.ones((1, eb), jnp.float32),
             jnp.zeros((16 - o - 1, eb), jnp.float32)], axis=0)

    return pl.pallas_call(
        body,
        grid=(grid,),
        in_specs=[
            pl.BlockSpec((d, eb), lambda i: (0, i)),
            pl.BlockSpec((xs_t.shape[0], eb), lambda i: (0, i)),
            pl.BlockSpec(w.shape, lambda i: (0, 0)),
            pl.BlockSpec(cvec.shape, lambda i: (0, 0)),
            pl.BlockSpec(rt.shape, lambda i: (0, 0)),
        ],
        out_specs=pl.BlockSpec((16, eb), lambda i: (0, i)),
        out_shape=jax.ShapeDtypeStruct((16, e), jnp.float32),
    )(pseudo_t, xs_t, w, cvec, rt)


def _tc_combine(agg2, x, root, bias):
    n, i_f = x.shape
    o = root.shape[0]
    nb = 1000
    grid = n // nb
    bias2 = bias[None, :]

    def body(a_ref, x_ref, root_ref, b_ref, out_ref):
        a = a_ref[...]
        s = a[0] + a[1]
        msg = s[:, 0:o]
        deg = s[:, o:o + 1]
        dense = lax.dot_general(x_ref[...], root_ref[...],
                                (((1,), (1,)), ((), ())),
                                preferred_element_type=jnp.float32)
        out_ref[...] = msg / jnp.maximum(deg, 1.0) + dense + b_ref[...]

    return pl.pallas_call(
        body,
        grid=(grid,),
        in_specs=[
            pl.BlockSpec((2, nb, 16), lambda i: (0, i, 0)),
            pl.BlockSpec((nb, i_f), lambda i: (i, 0)),
            pl.BlockSpec(root.shape, lambda i: (0, 0)),
            pl.BlockSpec((1, o), lambda i: (0, 0)),
        ],
        out_specs=pl.BlockSpec((nb, o), lambda i: (i, 0)),
        out_shape=jax.ShapeDtypeStruct((n, o), jnp.float32),
    )(agg2, x, root, bias2)


def kernel(edge_index, pseudo, x, mean, covariance, root, bias):
    e = edge_index.shape[1]
    n, i_f = x.shape
    o, _, k, d = mean.shape
    ew = e // NW
    nch = ew // CH
    src2 = edge_index[0].reshape(NW, ew)
    dst3 = edge_index[1].reshape(NW, nch, CH)

    mu = jnp.transpose(mean, (2, 0, 1, 3)).reshape(k * o * i_f, d)
    iv = 1.0 / (2.0 * jnp.transpose(covariance, (2, 0, 1, 3)
                                    ).reshape(k * o * i_f, d) ** 2 + 1e-8)
    w = jnp.concatenate([iv, -2.0 * mu * iv], axis=1)
    cvec = jnp.sum(mu * mu * iv, axis=1)[:, None]
    rt = jnp.repeat(jnp.eye(o, dtype=jnp.float32), i_f, axis=1) / k

    xs_t = _sc_gather(src2, x.T)
    msg16_t = _tc_dense(pseudo.T, xs_t, w, cvec, rt)
    agg2 = _sc_scatter(dst3, msg16_t, n)
    return _tc_combine(agg2, x, root, bias)


# speedup vs baseline: 1.0587x; 1.0587x over previous
import functools

import jax
import jax.numpy as jnp
from jax import lax
from jax.experimental import pallas as pl
from jax.experimental.pallas import tpu as pltpu
from jax.experimental.pallas import tpu_sc as plsc

NC = 2
NS = 16
NW = NC * NS
CH = 125


def _sc_gather(src2, xt):
    nw, ew = src2.shape
    f, n = xt.shape
    ewp = ((ew + 15) // 16) * 16
    ngr = ewp // 16
    mesh = plsc.VectorSubcoreMesh(core_axis_name="c", subcore_axis_name="s")

    @functools.partial(
        pl.kernel,
        out_type=jax.ShapeDtypeStruct((f, nw * ew), jnp.float32),
        mesh=mesh,
        compiler_params=pltpu.CompilerParams(use_tc_tiling_on_sc=False, needs_layout_passes=False),
        scratch_types=[
            pltpu.VMEM((f, n), jnp.float32),
            pltpu.VMEM((ewp,), jnp.int32),
            pltpu.VMEM((f, ewp), jnp.float32),
        ],
    )
    def gather_kernel(src_hbm, xt_hbm, xst_hbm, xt_v, idx_v, xst_v):
        wid = lax.axis_index("s") * NC + lax.axis_index("c")
        pltpu.sync_copy(xt_hbm, xt_v)
        pltpu.sync_copy(src_hbm.at[wid], idx_v.at[pl.ds(0, ew)])
        lanes = lax.iota(jnp.int32, 16)
        tail = idx_v[pl.ds(ewp - 16, 16)]
        idx_v[pl.ds(ewp - 16, 16)] = jnp.where(lanes < 16 - (ewp - ew),
                                               tail, 0)

        def body(q, carry):
            base = q * 16
            idx16 = idx_v[pl.ds(base, 16)]
            for ff in range(f):
                row = jnp.full((16,), ff, jnp.int32)
                xst_v[ff, pl.ds(base, 16)] = plsc.load_gather(
                    xt_v, [row, idx16])
            return carry

        lax.fori_loop(0, ngr, body, 0)
        pltpu.sync_copy(xst_v.at[:, pl.ds(0, ew)],
                        xst_hbm.at[:, pl.ds(wid * ew, ew)])

    return gather_kernel(src2, xt)


def _sc_scatter(dst3, msgt, n):
    nw, nch, ch = dst3.shape
    ew = nch * ch
    hch = 8
    nh = nch // hch
    hew = hch * ch
    unr = 25
    rpt = n // NS
    mesh = plsc.VectorSubcoreMesh(core_axis_name="c", subcore_axis_name="s")

    @functools.partial(
        pl.kernel,
        out_type=jax.ShapeDtypeStruct((NC, n, 16), jnp.float32),
        mesh=mesh,
        compiler_params=pltpu.CompilerParams(use_tc_tiling_on_sc=False, needs_layout_passes=False),
        scratch_types=[
            pltpu.VMEM((nch, ch), jnp.int32),
            pltpu.VMEM((16, hew), jnp.float32),
            pltpu.VMEM((hch, ch, 16), jnp.float32),
            pltpu.VMEM((rpt, 16), jnp.float32),
            pltpu.VMEM_SHARED((n, 16), jnp.float32),
            pltpu.SemaphoreType.DMA,
        ],
    )
    def scatter_kernel(dst_hbm, msgt_hbm, out_hbm, idx_v, msgt_v, msg_v,
                       zero_v, agg_sh, sem):
        cid = lax.axis_index("c")
        sid = lax.axis_index("s")
        wid = sid * NC + cid
        lanes = lax.iota(jnp.int32, 16)

        def zbody(i, carry):
            zero_v[i] = jnp.zeros((16,), jnp.float32)
            return carry

        lax.fori_loop(0, rpt, zbody, 0)
        pltpu.sync_copy(zero_v, agg_sh.at[pl.ds(sid * rpt, rpt)])
        pltpu.sync_copy(dst_hbm.at[wid], idx_v)
        plsc.subcore_barrier()

        for h in range(nh):
            pltpu.sync_copy(
                msgt_hbm.at[:, pl.ds(wid * ew + h * hew, hew)], msgt_v)

            def jbody(j, colv):
                def pbody(t, colv):
                    for u in range(unr):
                        p = t * unr + u
                        v = plsc.load_gather(msgt_v, [lanes, colv])
                        msg_v[j, p] = v
                        colv = colv + 1
                    return colv

                return lax.fori_loop(0, ch // unr, pbody, colv)

            lax.fori_loop(0, hch, jbody, lanes * 0)

            def sbody(j, carry):
                pltpu.async_copy(msg_v.at[j],
                                 agg_sh.at[idx_v.at[h * hch + j]],
                                 sem, add=True)
                return carry

            lax.fori_loop(0, hch, sbody, 0)

            def dbody(j, carry):
                pltpu.make_async_copy(msg_v.at[j],
                                      agg_sh.at[idx_v.at[h * hch + j]],
                                      sem).wait()
                return carry

            lax.fori_loop(0, hch, dbody, 0)

        plsc.subcore_barrier()
        pltpu.sync_copy(agg_sh.at[pl.ds(sid * rpt, rpt)],
                        out_hbm.at[cid, pl.ds(sid * rpt, rpt)])

    return scatter_kernel(dst3, msgt)


def _tc_dense(pseudo_t, xs_t, w, cvec, rt):
    d, e = pseudo_t.shape
    koi = w.shape[0]
    o, oi = rt.shape
    k = koi // oi
    eb = 3200
    grid = e // eb

    def body(p_ref, xs_ref, w_ref, c_ref, rt_ref, out_ref):
        p = p_ref[...]
        fmat = jnp.concatenate([p * p, p], axis=0)
        arg = lax.dot_general(w_ref[...], fmat,
                              (((1,), (0,)), ((), ())),
                              preferred_element_type=jnp.float32)
        g = jnp.exp(-(arg + c_ref[...]))
        gm = g[0:oi]
        for kk in range(1, k):
            gm = gm + g[kk * oi:(kk + 1) * oi]
        prod = gm * jnp.tile(xs_ref[...], (o, 1))
        msg = lax.dot_general(rt_ref[...], prod,
                              (((1,), (0,)), ((), ())),
                              preferred_element_type=jnp.float32)
        out_ref[...] = jnp.concatenate(
            [msg,
             jnp.ones((1, eb), jnp.float32),
             jnp.zeros((16 - o - 1, eb), jnp.float32)], axis=0)

    return pl.pallas_call(
        body,
        grid=(grid,),
        in_specs=[
            pl.BlockSpec((d, eb), lambda i: (0, i)),
            pl.BlockSpec((xs_t.shape[0], eb), lambda i: (0, i)),
            pl.BlockSpec(w.shape, lambda i: (0, 0)),
            pl.BlockSpec(cvec.shape, lambda i: (0, 0)),
            pl.BlockSpec(rt.shape, lambda i: (0, 0)),
        ],
        out_specs=pl.BlockSpec((16, eb), lambda i: (0, i)),
        out_shape=jax.ShapeDtypeStruct((16, e), jnp.float32),
    )(pseudo_t, xs_t, w, cvec, rt)


def _tc_combine(agg2, x, root, bias):
    n, i_f = x.shape
    o = root.shape[0]
    nb = 1000
    grid = n // nb
    bias2 = bias[None, :]

    def body(a_ref, x_ref, root_ref, b_ref, out_ref):
        a = a_ref[...]
        s = a[0] + a[1]
        msg = s[:, 0:o]
        deg = s[:, o:o + 1]
        dense = lax.dot_general(x_ref[...], root_ref[...],
                                (((1,), (1,)), ((), ())),
                                preferred_element_type=jnp.float32)
        out_ref[...] = msg / jnp.maximum(deg, 1.0) + dense + b_ref[...]

    return pl.pallas_call(
        body,
        grid=(grid,),
        in_specs=[
            pl.BlockSpec((2, nb, 16), lambda i: (0, i, 0)),
            pl.BlockSpec((nb, i_f), lambda i: (i, 0)),
            pl.BlockSpec(root.shape, lambda i: (0, 0)),
            pl.BlockSpec((1, o), lambda i: (0, 0)),
        ],
        out_specs=pl.BlockSpec((nb, o), lambda i: (i, 0)),
        out_shape=jax.ShapeDtypeStruct((n, o), jnp.float32),
    )(agg2, x, root, bias2)


def kernel(edge_index, pseudo, x, mean, covariance, root, bias):
    e = edge_index.shape[1]
    n, i_f = x.shape
    o, _, k, d = mean.shape
    ew = e // NW
    nch = ew // CH
    src2 = edge_index[0].reshape(NW, ew)
    dst3 = edge_index[1].reshape(NW, nch, CH)

    mu = jnp.transpose(mean, (2, 0, 1, 3)).reshape(k * o * i_f, d)
    iv = 1.0 / (2.0 * jnp.transpose(covariance, (2, 0, 1, 3)
                                    ).reshape(k * o * i_f, d) ** 2 + 1e-8)
    w = jnp.concatenate([iv, -2.0 * mu * iv], axis=1)
    cvec = jnp.sum(mu * mu * iv, axis=1)[:, None]
    rt = jnp.repeat(jnp.eye(o, dtype=jnp.float32), i_f, axis=1) / k

    xs_t = _sc_gather(src2, x.T)
    msg16_t = _tc_dense(pseudo.T, xs_t, w, cvec, rt)
    agg2 = _sc_scatter(dst3, msg16_t, n)
    return _tc_combine(agg2, x, root, bias)


# speedup vs baseline: 1.1840x; 1.1184x over previous
import functools

import jax
import jax.numpy as jnp
from jax import lax
from jax.experimental import pallas as pl
from jax.experimental.pallas import tpu as pltpu
from jax.experimental.pallas import tpu_sc as plsc

NC = 2
NS = 16
NW = NC * NS
CH = 125


def _sc_gather(src2, xt):
    nw, ew = src2.shape
    f, n = xt.shape
    ewp = ((ew + 15) // 16) * 16
    ngr = ewp // 16
    mesh = plsc.VectorSubcoreMesh(core_axis_name="c", subcore_axis_name="s")

    @functools.partial(
        pl.kernel,
        out_type=jax.ShapeDtypeStruct((f, nw * ew), jnp.float32),
        mesh=mesh,
        compiler_params=pltpu.CompilerParams(use_tc_tiling_on_sc=False, needs_layout_passes=False),
        scratch_types=[
            pltpu.VMEM((f, n), jnp.float32),
            pltpu.VMEM((ewp,), jnp.int32),
            pltpu.VMEM((f, ewp), jnp.float32),
        ],
    )
    def gather_kernel(src_hbm, xt_hbm, xst_hbm, xt_v, idx_v, xst_v):
        wid = lax.axis_index("s") * NC + lax.axis_index("c")
        pltpu.sync_copy(xt_hbm, xt_v)
        pltpu.sync_copy(src_hbm.at[wid], idx_v.at[pl.ds(0, ew)])
        lanes = lax.iota(jnp.int32, 16)
        tail = idx_v[pl.ds(ewp - 16, 16)]
        idx_v[pl.ds(ewp - 16, 16)] = jnp.where(lanes < 16 - (ewp - ew),
                                               tail, 0)

        @plsc.parallel_loop(0, ngr, unroll=4)
        def _(q):
            base = q * 16
            idx16 = idx_v[pl.ds(base, 16)]
            for ff in range(f):
                row = jnp.full((16,), ff, jnp.int32)
                xst_v[ff, pl.ds(base, 16)] = plsc.load_gather(
                    xt_v, [row, idx16])
        pltpu.sync_copy(xst_v.at[:, pl.ds(0, ew)],
                        xst_hbm.at[:, pl.ds(wid * ew, ew)])

    return gather_kernel(src2, xt)


def _sc_scatter(dst3, msgt, n):
    nw, nch, ch = dst3.shape
    ew = nch * ch
    hch = 8
    nh = nch // hch
    hew = hch * ch
    unr = 8
    rpt = n // NS
    mesh = plsc.VectorSubcoreMesh(core_axis_name="c", subcore_axis_name="s")

    @functools.partial(
        pl.kernel,
        out_type=jax.ShapeDtypeStruct((NC, n, 16), jnp.float32),
        mesh=mesh,
        compiler_params=pltpu.CompilerParams(use_tc_tiling_on_sc=False, needs_layout_passes=False),
        scratch_types=[
            pltpu.VMEM((nch, ch), jnp.int32),
            pltpu.VMEM((16, hew), jnp.float32),
            pltpu.VMEM((hch, ch, 16), jnp.float32),
            pltpu.VMEM((rpt, 16), jnp.float32),
            pltpu.VMEM_SHARED((n, 16), jnp.float32),
            pltpu.SemaphoreType.DMA,
        ],
    )
    def scatter_kernel(dst_hbm, msgt_hbm, out_hbm, idx_v, msgt_v, msg_v,
                       zero_v, agg_sh, sem):
        cid = lax.axis_index("c")
        sid = lax.axis_index("s")
        wid = sid * NC + cid
        lanes = lax.iota(jnp.int32, 16)

        def zbody(i, carry):
            zero_v[i] = jnp.zeros((16,), jnp.float32)
            return carry

        lax.fori_loop(0, rpt, zbody, 0)
        pltpu.sync_copy(zero_v, agg_sh.at[pl.ds(sid * rpt, rpt)])
        pltpu.sync_copy(dst_hbm.at[wid], idx_v)
        plsc.subcore_barrier()

        for h in range(nh):
            pltpu.sync_copy(
                msgt_hbm.at[:, pl.ds(wid * ew + h * hew, hew)], msgt_v)

            @plsc.parallel_loop(0, hch * ch, unroll=unr)
            def _(e):
                col = jnp.full((16,), e, jnp.int32)
                v = plsc.load_gather(msgt_v, [lanes, col])
                msg_v[e // ch, e % ch] = v

            def sbody(j, carry):
                pltpu.async_copy(msg_v.at[j],
                                 agg_sh.at[idx_v.at[h * hch + j]],
                                 sem, add=True)
                return carry

            lax.fori_loop(0, hch, sbody, 0)

            def dbody(j, carry):
                pltpu.make_async_copy(msg_v.at[j],
                                      agg_sh.at[idx_v.at[h * hch + j]],
                                      sem).wait()
                return carry

            lax.fori_loop(0, hch, dbody, 0)

        plsc.subcore_barrier()
        pltpu.sync_copy(agg_sh.at[pl.ds(sid * rpt, rpt)],
                        out_hbm.at[cid, pl.ds(sid * rpt, rpt)])

    return scatter_kernel(dst3, msgt)


def _tc_dense(pseudo_t, xs_t, w, cvec, rt):
    d, e = pseudo_t.shape
    koi = w.shape[0]
    o, oi = rt.shape
    k = koi // oi
    eb = 3200
    grid = e // eb

    def body(p_ref, xs_ref, w_ref, c_ref, rt_ref, out_ref):
        p = p_ref[...]
        fmat = jnp.concatenate([p * p, p], axis=0)
        arg = lax.dot_general(w_ref[...], fmat,
                              (((1,), (0,)), ((), ())),
                              preferred_element_type=jnp.float32)
        g = jnp.exp(-(arg + c_ref[...]))
        gm = g[0:oi]
        for kk in range(1, k):
            gm = gm + g[kk * oi:(kk + 1) * oi]
        prod = gm * jnp.tile(xs_ref[...], (o, 1))
        msg = lax.dot_general(rt_ref[...], prod,
                              (((1,), (0,)), ((), ())),
                              preferred_element_type=jnp.float32)
        out_ref[...] = jnp.concatenate(
            [msg,
             jnp.ones((1, eb), jnp.float32),
             jnp.zeros((16 - o - 1, eb), jnp.float32)], axis=0)

    return pl.pallas_call(
        body,
        grid=(grid,),
        in_specs=[
            pl.BlockSpec((d, eb), lambda i: (0, i)),
            pl.BlockSpec((xs_t.shape[0], eb), lambda i: (0, i)),
            pl.BlockSpec(w.shape, lambda i: (0, 0)),
            pl.BlockSpec(cvec.shape, lambda i: (0, 0)),
            pl.BlockSpec(rt.shape, lambda i: (0, 0)),
        ],
        out_specs=pl.BlockSpec((16, eb), lambda i: (0, i)),
        out_shape=jax.ShapeDtypeStruct((16, e), jnp.float32),
    )(pseudo_t, xs_t, w, cvec, rt)


def _tc_combine(agg2, x, root, bias):
    n, i_f = x.shape
    o = root.shape[0]
    nb = 1000
    grid = n // nb
    bias2 = bias[None, :]

    def body(a_ref, x_ref, root_ref, b_ref, out_ref):
        a = a_ref[...]
        s = a[0] + a[1]
        msg = s[:, 0:o]
        deg = s[:, o:o + 1]
        dense = lax.dot_general(x_ref[...], root_ref[...],
                                (((1,), (1,)), ((), ())),
                                preferred_element_type=jnp.float32)
        out_ref[...] = msg / jnp.maximum(deg, 1.0) + dense + b_ref[...]

    return pl.pallas_call(
        body,
        grid=(grid,),
        in_specs=[
            pl.BlockSpec((2, nb, 16), lambda i: (0, i, 0)),
            pl.BlockSpec((nb, i_f), lambda i: (i, 0)),
            pl.BlockSpec(root.shape, lambda i: (0, 0)),
            pl.BlockSpec((1, o), lambda i: (0, 0)),
        ],
        out_specs=pl.BlockSpec((nb, o), lambda i: (i, 0)),
        out_shape=jax.ShapeDtypeStruct((n, o), jnp.float32),
    )(agg2, x, root, bias2)


def kernel(edge_index, pseudo, x, mean, covariance, root, bias):
    e = edge_index.shape[1]
    n, i_f = x.shape
    o, _, k, d = mean.shape
    ew = e // NW
    nch = ew // CH
    src2 = edge_index[0].reshape(NW, ew)
    dst3 = edge_index[1].reshape(NW, nch, CH)

    mu = jnp.transpose(mean, (2, 0, 1, 3)).reshape(k * o * i_f, d)
    iv = 1.0 / (2.0 * jnp.transpose(covariance, (2, 0, 1, 3)
                                    ).reshape(k * o * i_f, d) ** 2 + 1e-8)
    w = jnp.concatenate([iv, -2.0 * mu * iv], axis=1)
    cvec = jnp.sum(mu * mu * iv, axis=1)[:, None]
    rt = jnp.repeat(jnp.eye(o, dtype=jnp.float32), i_f, axis=1) / k

    xs_t = _sc_gather(src2, x.T)
    msg16_t = _tc_dense(pseudo.T, xs_t, w, cvec, rt)
    agg2 = _sc_scatter(dst3, msg16_t, n)
    return _tc_combine(agg2, x, root, bias)


# speedup vs baseline: 1.3004x; 1.0983x over previous
import functools

import jax
import jax.numpy as jnp
from jax import lax
from jax.experimental import pallas as pl
from jax.experimental.pallas import tpu as pltpu
from jax.experimental.pallas import tpu_sc as plsc

NC = 2
NS = 16
NW = NC * NS
CH = 125


def _sc_gather(src2, xt):
    nw, ew = src2.shape
    f, n = xt.shape
    ewp = ((ew + 15) // 16) * 16
    ngr = ewp // 16
    mesh = plsc.VectorSubcoreMesh(core_axis_name="c", subcore_axis_name="s")

    @functools.partial(
        pl.kernel,
        out_type=jax.ShapeDtypeStruct((f, nw * ew), jnp.float32),
        mesh=mesh,
        compiler_params=pltpu.CompilerParams(use_tc_tiling_on_sc=False, needs_layout_passes=False),
        scratch_types=[
            pltpu.VMEM((f, n), jnp.float32),
            pltpu.VMEM((ewp,), jnp.int32),
            pltpu.VMEM((f, ewp), jnp.float32),
        ],
    )
    def gather_kernel(src_hbm, xt_hbm, xst_hbm, xt_v, idx_v, xst_v):
        wid = lax.axis_index("s") * NC + lax.axis_index("c")
        pltpu.sync_copy(xt_hbm, xt_v)
        pltpu.sync_copy(src_hbm.at[wid], idx_v.at[pl.ds(0, ew)])
        lanes = lax.iota(jnp.int32, 16)
        tail = idx_v[pl.ds(ewp - 16, 16)]
        idx_v[pl.ds(ewp - 16, 16)] = jnp.where(lanes < 16 - (ewp - ew),
                                               tail, 0)

        @plsc.parallel_loop(0, ngr, unroll=4)
        def _(q):
            base = q * 16
            idx16 = idx_v[pl.ds(base, 16)]
            for ff in range(f):
                row = jnp.full((16,), ff, jnp.int32)
                xst_v[ff, pl.ds(base, 16)] = plsc.load_gather(
                    xt_v, [row, idx16])
        pltpu.sync_copy(xst_v.at[:, pl.ds(0, ew)],
                        xst_hbm.at[:, pl.ds(wid * ew, ew)])

    return gather_kernel(src2, xt)


def _sc_scatter(dst3, msgt, n):
    nw, nch, ch = dst3.shape
    ew = nch * ch
    hch = 8
    nh = nch // hch
    hew = hch * ch
    unr = 8
    rpt = n // NS
    mesh = plsc.VectorSubcoreMesh(core_axis_name="c", subcore_axis_name="s")

    @functools.partial(
        pl.kernel,
        out_type=jax.ShapeDtypeStruct((NC, n, 16), jnp.float32),
        mesh=mesh,
        compiler_params=pltpu.CompilerParams(use_tc_tiling_on_sc=False, needs_layout_passes=False),
        scratch_types=[
            pltpu.VMEM((nch, ch), jnp.int32),
            pltpu.VMEM((16, hew), jnp.float32),
            pltpu.VMEM((hch, ch, 16), jnp.float32),
            pltpu.VMEM((rpt, 16), jnp.float32),
            pltpu.VMEM_SHARED((n, 16), jnp.float32),
            pltpu.SemaphoreType.DMA,
        ],
    )
    def scatter_kernel(dst_hbm, msgt_hbm, out_hbm, idx_v, msgt_v, msg_v,
                       zero_v, agg_sh, sem):
        cid = lax.axis_index("c")
        sid = lax.axis_index("s")
        wid = sid * NC + cid
        lanes = lax.iota(jnp.int32, 16)

        def zbody(i, carry):
            zero_v[i] = jnp.zeros((16,), jnp.float32)
            return carry

        lax.fori_loop(0, rpt, zbody, 0)
        pltpu.sync_copy(zero_v, agg_sh.at[pl.ds(sid * rpt, rpt)])
        pltpu.sync_copy(dst_hbm.at[wid], idx_v)
        plsc.subcore_barrier()

        for h in range(nh):
            pltpu.sync_copy(
                msgt_hbm.at[:, pl.ds(wid * ew + h * hew, hew)], msgt_v)

            @plsc.parallel_loop(0, hch * ch, unroll=unr)
            def _(e):
                col = jnp.full((16,), e, jnp.int32)
                v = plsc.load_gather(msgt_v, [lanes, col])
                msg_v[e // ch, e % ch] = v

            def sbody(j, carry):
                pltpu.async_copy(msg_v.at[j],
                                 agg_sh.at[idx_v.at[h * hch + j]],
                                 sem, add=True)
                return carry

            lax.fori_loop(0, hch, sbody, 0)

            def dbody(j, carry):
                pltpu.make_async_copy(msg_v.at[j],
                                      agg_sh.at[idx_v.at[h * hch + j]],
                                      sem).wait()
                return carry

            lax.fori_loop(0, hch, dbody, 0)

        plsc.subcore_barrier()
        pltpu.sync_copy(agg_sh.at[pl.ds(sid * rpt, rpt)],
                        out_hbm.at[cid, pl.ds(sid * rpt, rpt)])

    return scatter_kernel(dst3, msgt)


def _tc_dense(pseudo_t, xs_t, w, cvec, rt):
    d, e = pseudo_t.shape
    koi = w.shape[0]
    o, oi = rt.shape
    k = koi // oi
    eb = 6400
    grid = e // eb

    def body(p_ref, xs_ref, w_ref, c_ref, rt_ref, out_ref):
        p = p_ref[...]
        fmat = jnp.concatenate([p * p, p], axis=0)
        arg = lax.dot_general(w_ref[...], fmat,
                              (((1,), (0,)), ((), ())),
                              preferred_element_type=jnp.float32)
        g = jnp.exp(-(arg + c_ref[...]))
        gm = g[0:oi]
        for kk in range(1, k):
            gm = gm + g[kk * oi:(kk + 1) * oi]
        prod = gm * jnp.tile(xs_ref[...], (o, 1))
        msg = lax.dot_general(rt_ref[...], prod,
                              (((1,), (0,)), ((), ())),
                              preferred_element_type=jnp.float32)
        out_ref[...] = jnp.concatenate(
            [msg,
             jnp.ones((1, eb), jnp.float32),
             jnp.zeros((16 - o - 1, eb), jnp.float32)], axis=0)

    return pl.pallas_call(
        body,
        grid=(grid,),
        in_specs=[
            pl.BlockSpec((d, eb), lambda i: (0, i)),
            pl.BlockSpec((xs_t.shape[0], eb), lambda i: (0, i)),
            pl.BlockSpec(w.shape, lambda i: (0, 0)),
            pl.BlockSpec(cvec.shape, lambda i: (0, 0)),
            pl.BlockSpec(rt.shape, lambda i: (0, 0)),
        ],
        out_specs=pl.BlockSpec((16, eb), lambda i: (0, i)),
        out_shape=jax.ShapeDtypeStruct((16, e), jnp.float32),
    )(pseudo_t, xs_t, w, cvec, rt)


def _tc_combine(agg2, x, root, bias):
    n, i_f = x.shape
    o = root.shape[0]
    nb = 1000
    grid = n // nb
    bias2 = bias[None, :]

    def body(a_ref, x_ref, root_ref, b_ref, out_ref):
        a = a_ref[...]
        s = a[0] + a[1]
        msg = s[:, 0:o]
        deg = s[:, o:o + 1]
        dense = lax.dot_general(x_ref[...], root_ref[...],
                                (((1,), (1,)), ((), ())),
                                preferred_element_type=jnp.float32)
        out_ref[...] = msg / jnp.maximum(deg, 1.0) + dense + b_ref[...]

    return pl.pallas_call(
        body,
        grid=(grid,),
        in_specs=[
            pl.BlockSpec((2, nb, 16), lambda i: (0, i, 0)),
            pl.BlockSpec((nb, i_f), lambda i: (i, 0)),
            pl.BlockSpec(root.shape, lambda i: (0, 0)),
            pl.BlockSpec((1, o), lambda i: (0, 0)),
        ],
        out_specs=pl.BlockSpec((nb, o), lambda i: (i, 0)),
        out_shape=jax.ShapeDtypeStruct((n, o), jnp.float32),
    )(agg2, x, root, bias2)


def kernel(edge_index, pseudo, x, mean, covariance, root, bias):
    e = edge_index.shape[1]
    n, i_f = x.shape
    o, _, k, d = mean.shape
    ew = e // NW
    nch = ew // CH
    src2 = edge_index[0].reshape(NW, ew)
    dst3 = edge_index[1].reshape(NW, nch, CH)

    mu = jnp.transpose(mean, (2, 0, 1, 3)).reshape(k * o * i_f, d)
    iv = 1.0 / (2.0 * jnp.transpose(covariance, (2, 0, 1, 3)
                                    ).reshape(k * o * i_f, d) ** 2 + 1e-8)
    w = jnp.concatenate([iv, -2.0 * mu * iv], axis=1)
    cvec = jnp.sum(mu * mu * iv, axis=1)[:, None]
    rt = jnp.repeat(jnp.eye(o, dtype=jnp.float32), i_f, axis=1) / k

    xs_t = _sc_gather(src2, x.T)
    msg16_t = _tc_dense(pseudo.T, xs_t, w, cvec, rt)
    agg2 = _sc_scatter(dst3, msg16_t, n)
    return _tc_combine(agg2, x, root, bias)


# speedup vs baseline: 1.3542x; 1.0413x over previous
import functools

import jax
import jax.numpy as jnp
from jax import lax
from jax.experimental import pallas as pl
from jax.experimental.pallas import tpu as pltpu
from jax.experimental.pallas import tpu_sc as plsc

NC = 2
NS = 16
NW = NC * NS
CH = 125


def _sc_gather(src2, xt):
    nw, ew = src2.shape
    f, n = xt.shape
    ewp = ((ew + 15) // 16) * 16
    ngr = ewp // 16
    mesh = plsc.VectorSubcoreMesh(core_axis_name="c", subcore_axis_name="s")

    @functools.partial(
        pl.kernel,
        out_type=jax.ShapeDtypeStruct((f, nw * ew), jnp.float32),
        mesh=mesh,
        compiler_params=pltpu.CompilerParams(use_tc_tiling_on_sc=False, needs_layout_passes=False),
        scratch_types=[
            pltpu.VMEM((f, n), jnp.float32),
            pltpu.VMEM((ewp,), jnp.int32),
            pltpu.VMEM((f, ewp), jnp.float32),
        ],
    )
    def gather_kernel(src_hbm, xt_hbm, xst_hbm, xt_v, idx_v, xst_v):
        wid = lax.axis_index("s") * NC + lax.axis_index("c")
        pltpu.sync_copy(xt_hbm, xt_v)
        pltpu.sync_copy(src_hbm.at[wid], idx_v.at[pl.ds(0, ew)])
        lanes = lax.iota(jnp.int32, 16)
        tail = idx_v[pl.ds(ewp - 16, 16)]
        idx_v[pl.ds(ewp - 16, 16)] = jnp.where(lanes < 16 - (ewp - ew),
                                               tail, 0)

        @plsc.parallel_loop(0, ngr, unroll=4)
        def _(q):
            base = q * 16
            idx16 = idx_v[pl.ds(base, 16)]
            for ff in range(f):
                row = jnp.full((16,), ff, jnp.int32)
                xst_v[ff, pl.ds(base, 16)] = plsc.load_gather(
                    xt_v, [row, idx16])
        pltpu.sync_copy(xst_v.at[:, pl.ds(0, ew)],
                        xst_hbm.at[:, pl.ds(wid * ew, ew)])

    return gather_kernel(src2, xt)


def _sc_scatter(dst3, msgt, n):
    nw, nch, ch = dst3.shape
    ew = nch * ch
    hch = 8
    nh = nch // hch
    hew = hch * ch
    unr = 8
    rpt = n // NS
    mesh = plsc.VectorSubcoreMesh(core_axis_name="c", subcore_axis_name="s")

    @functools.partial(
        pl.kernel,
        out_type=jax.ShapeDtypeStruct((NC, n, 16), jnp.float32),
        mesh=mesh,
        compiler_params=pltpu.CompilerParams(use_tc_tiling_on_sc=False, needs_layout_passes=False),
        scratch_types=[
            pltpu.VMEM((nch, ch), jnp.int32),
            pltpu.VMEM((16, hew), jnp.float32),
            pltpu.VMEM((hch, ch, 16), jnp.float32),
            pltpu.VMEM((rpt, 16), jnp.float32),
            pltpu.VMEM_SHARED((n, 16), jnp.float32),
            pltpu.SemaphoreType.DMA,
        ],
    )
    def scatter_kernel(dst_hbm, msgt_hbm, out_hbm, idx_v, msgt_v, msg_v,
                       zero_v, agg_sh, sem):
        cid = lax.axis_index("c")
        sid = lax.axis_index("s")
        wid = sid * NC + cid
        lanes = lax.iota(jnp.int32, 16)

        def zbody(i, carry):
            zero_v[i] = jnp.zeros((16,), jnp.float32)
            return carry

        lax.fori_loop(0, rpt, zbody, 0)
        pltpu.sync_copy(zero_v, agg_sh.at[pl.ds(sid * rpt, rpt)])
        pltpu.sync_copy(dst_hbm.at[wid], idx_v)
        plsc.subcore_barrier()

        for h in range(nh):
            pltpu.sync_copy(
                msgt_hbm.at[:, pl.ds(wid * ew + h * hew, hew)], msgt_v)

            @plsc.parallel_loop(0, hch * ch, unroll=unr)
            def _(e):
                col = jnp.full((16,), e, jnp.int32)
                v = plsc.load_gather(msgt_v, [lanes, col])
                msg_v[e // ch, e % ch] = v

            def sbody(j, carry):
                pltpu.async_copy(msg_v.at[j],
                                 agg_sh.at[idx_v.at[h * hch + j]],
                                 sem, add=True)
                return carry

            lax.fori_loop(0, hch, sbody, 0)

            def dbody(j, carry):
                pltpu.make_async_copy(msg_v.at[j],
                                      agg_sh.at[idx_v.at[h * hch + j]],
                                      sem).wait()
                return carry

            lax.fori_loop(0, hch, dbody, 0)

        plsc.subcore_barrier()
        pltpu.sync_copy(agg_sh.at[pl.ds(sid * rpt, rpt)],
                        out_hbm.at[cid, pl.ds(sid * rpt, rpt)])

    return scatter_kernel(dst3, msgt)


def _tc_dense(pseudo_t, xs_t, w, cvec, rt):
    d, e = pseudo_t.shape
    koi = w.shape[0]
    o, oi = rt.shape
    k = koi // oi
    eb = 16000
    grid = e // eb

    def body(p_ref, xs_ref, w_ref, c_ref, rt_ref, out_ref):
        p = p_ref[...]
        fmat = jnp.concatenate([p * p, p], axis=0)
        arg = lax.dot_general(w_ref[...], fmat,
                              (((1,), (0,)), ((), ())),
                              preferred_element_type=jnp.float32)
        g = jnp.exp(-(arg + c_ref[...]))
        gm = g[0:oi]
        for kk in range(1, k):
            gm = gm + g[kk * oi:(kk + 1) * oi]
        prod = gm * jnp.tile(xs_ref[...], (o, 1))
        msg = lax.dot_general(rt_ref[...], prod,
                              (((1,), (0,)), ((), ())),
                              preferred_element_type=jnp.float32)
        out_ref[...] = jnp.concatenate(
            [msg,
             jnp.ones((1, eb), jnp.float32),
             jnp.zeros((16 - o - 1, eb), jnp.float32)], axis=0)

    return pl.pallas_call(
        body,
        grid=(grid,),
        in_specs=[
            pl.BlockSpec((d, eb), lambda i: (0, i)),
            pl.BlockSpec((xs_t.shape[0], eb), lambda i: (0, i)),
            pl.BlockSpec(w.shape, lambda i: (0, 0)),
            pl.BlockSpec(cvec.shape, lambda i: (0, 0)),
            pl.BlockSpec(rt.shape, lambda i: (0, 0)),
        ],
        out_specs=pl.BlockSpec((16, eb), lambda i: (0, i)),
        out_shape=jax.ShapeDtypeStruct((16, e), jnp.float32),
    )(pseudo_t, xs_t, w, cvec, rt)


def _tc_combine(agg2, x, root, bias):
    n, i_f = x.shape
    o = root.shape[0]
    nb = 1000
    grid = n // nb
    bias2 = bias[None, :]

    def body(a_ref, x_ref, root_ref, b_ref, out_ref):
        a = a_ref[...]
        s = a[0] + a[1]
        msg = s[:, 0:o]
        deg = s[:, o:o + 1]
        dense = lax.dot_general(x_ref[...], root_ref[...],
                                (((1,), (1,)), ((), ())),
                                preferred_element_type=jnp.float32)
        out_ref[...] = msg / jnp.maximum(deg, 1.0) + dense + b_ref[...]

    return pl.pallas_call(
        body,
        grid=(grid,),
        in_specs=[
            pl.BlockSpec((2, nb, 16), lambda i: (0, i, 0)),
            pl.BlockSpec((nb, i_f), lambda i: (i, 0)),
            pl.BlockSpec(root.shape, lambda i: (0, 0)),
            pl.BlockSpec((1, o), lambda i: (0, 0)),
        ],
        out_specs=pl.BlockSpec((nb, o), lambda i: (i, 0)),
        out_shape=jax.ShapeDtypeStruct((n, o), jnp.float32),
    )(agg2, x, root, bias2)


def kernel(edge_index, pseudo, x, mean, covariance, root, bias):
    e = edge_index.shape[1]
    n, i_f = x.shape
    o, _, k, d = mean.shape
    ew = e // NW
    nch = ew // CH
    src2 = edge_index[0].reshape(NW, ew)
    dst3 = edge_index[1].reshape(NW, nch, CH)

    mu = jnp.transpose(mean, (2, 0, 1, 3)).reshape(k * o * i_f, d)
    iv = 1.0 / (2.0 * jnp.transpose(covariance, (2, 0, 1, 3)
                                    ).reshape(k * o * i_f, d) ** 2 + 1e-8)
    w = jnp.concatenate([iv, -2.0 * mu * iv], axis=1)
    cvec = jnp.sum(mu * mu * iv, axis=1)[:, None]
    rt = jnp.repeat(jnp.eye(o, dtype=jnp.float32), i_f, axis=1) / k

    xs_t = _sc_gather(src2, x.T)
    msg16_t = _tc_dense(pseudo.T, xs_t, w, cvec, rt)
    agg2 = _sc_scatter(dst3, msg16_t, n)
    return _tc_combine(agg2, x, root, bias)


# speedup vs baseline: 1.3586x; 1.0033x over previous
import functools

import jax
import jax.numpy as jnp
from jax import lax
from jax.experimental import pallas as pl
from jax.experimental.pallas import tpu as pltpu
from jax.experimental.pallas import tpu_sc as plsc

NC = 2
NS = 16
NW = NC * NS
CH = 125


def _sc_gather(src2, xt):
    nw, ew = src2.shape
    f, n = xt.shape
    ewp = ((ew + 15) // 16) * 16
    ngr = ewp // 16
    mesh = plsc.VectorSubcoreMesh(core_axis_name="c", subcore_axis_name="s")

    @functools.partial(
        pl.kernel,
        out_type=jax.ShapeDtypeStruct((f, nw * ew), jnp.float32),
        mesh=mesh,
        compiler_params=pltpu.CompilerParams(use_tc_tiling_on_sc=False, needs_layout_passes=False),
        scratch_types=[
            pltpu.VMEM((f, n), jnp.float32),
            pltpu.VMEM((ewp,), jnp.int32),
            pltpu.VMEM((f, ewp), jnp.float32),
        ],
    )
    def gather_kernel(src_hbm, xt_hbm, xst_hbm, xt_v, idx_v, xst_v):
        wid = lax.axis_index("s") * NC + lax.axis_index("c")
        pltpu.sync_copy(xt_hbm, xt_v)
        pltpu.sync_copy(src_hbm.at[wid], idx_v.at[pl.ds(0, ew)])
        lanes = lax.iota(jnp.int32, 16)
        tail = idx_v[pl.ds(ewp - 16, 16)]
        idx_v[pl.ds(ewp - 16, 16)] = jnp.where(lanes < 16 - (ewp - ew),
                                               tail, 0)

        @plsc.parallel_loop(0, ngr, unroll=4)
        def _(q):
            base = q * 16
            idx16 = idx_v[pl.ds(base, 16)]
            for ff in range(f):
                row = jnp.full((16,), ff, jnp.int32)
                xst_v[ff, pl.ds(base, 16)] = plsc.load_gather(
                    xt_v, [row, idx16])
        pltpu.sync_copy(xst_v.at[:, pl.ds(0, ew)],
                        xst_hbm.at[:, pl.ds(wid * ew, ew)])

    return gather_kernel(src2, xt)


def _sc_scatter(dst3, msgt, n):
    nw, nch, ch = dst3.shape
    ew = nch * ch
    hch = 8
    nh = nch // hch
    hew = hch * ch
    unr = 8
    rpt = n // NS
    mesh = plsc.VectorSubcoreMesh(core_axis_name="c", subcore_axis_name="s")

    @functools.partial(
        pl.kernel,
        out_type=jax.ShapeDtypeStruct((NC, n, 16), jnp.float32),
        mesh=mesh,
        compiler_params=pltpu.CompilerParams(use_tc_tiling_on_sc=False, needs_layout_passes=False),
        scratch_types=[
            pltpu.VMEM((nch, ch), jnp.int32),
            pltpu.VMEM((16, hew), jnp.float32),
            pltpu.VMEM((hch, ch, 16), jnp.float32),
            pltpu.VMEM((rpt, 16), jnp.float32),
            pltpu.VMEM_SHARED((n, 16), jnp.float32),
            pltpu.SemaphoreType.DMA,
        ],
    )
    def scatter_kernel(dst_hbm, msgt_hbm, out_hbm, idx_v, msgt_v, msg_v,
                       zero_v, agg_sh, sem):
        cid = lax.axis_index("c")
        sid = lax.axis_index("s")
        wid = sid * NC + cid
        lanes = lax.iota(jnp.int32, 16)

        def zbody(i, carry):
            zero_v[i] = jnp.zeros((16,), jnp.float32)
            return carry

        lax.fori_loop(0, rpt, zbody, 0)
        pltpu.sync_copy(zero_v, agg_sh.at[pl.ds(sid * rpt, rpt)])
        pltpu.sync_copy(dst_hbm.at[wid], idx_v)
        plsc.subcore_barrier()

        for h in range(nh):
            pltpu.sync_copy(
                msgt_hbm.at[:, pl.ds(wid * ew + h * hew, hew)], msgt_v)

            @plsc.parallel_loop(0, hch * ch, unroll=unr)
            def _(e):
                col = jnp.full((16,), e, jnp.int32)
                v = plsc.load_gather(msgt_v, [lanes, col])
                msg_v[e // ch, e % ch] = v

            def sbody(j, carry):
                pltpu.async_copy(msg_v.at[j],
                                 agg_sh.at[idx_v.at[h * hch + j]],
                                 sem, add=True)
                return carry

            lax.fori_loop(0, hch, sbody, 0)

            def dbody(j, carry):
                pltpu.make_async_copy(msg_v.at[j],
                                      agg_sh.at[idx_v.at[h * hch + j]],
                                      sem).wait()
                return carry

            lax.fori_loop(0, hch, dbody, 0)

        plsc.subcore_barrier()
        pltpu.sync_copy(agg_sh.at[pl.ds(sid * rpt, rpt)],
                        out_hbm.at[cid, pl.ds(sid * rpt, rpt)])

    return scatter_kernel(dst3, msgt)


def _tc_dense(pseudo_t, xs_t, w, cvec, rt):
    d, e = pseudo_t.shape
    koi = w.shape[0]
    o, oi = rt.shape
    k = koi // oi
    eb = 32000
    grid = e // eb

    def body(p_ref, xs_ref, w_ref, c_ref, rt_ref, out_ref):
        p = p_ref[...]
        fmat = jnp.concatenate([p * p, p], axis=0)
        arg = lax.dot_general(w_ref[...], fmat,
                              (((1,), (0,)), ((), ())),
                              preferred_element_type=jnp.float32)
        g = jnp.exp(-(arg + c_ref[...]))
        gm = g[0:oi]
        for kk in range(1, k):
            gm = gm + g[kk * oi:(kk + 1) * oi]
        prod = gm * jnp.tile(xs_ref[...], (o, 1))
        msg = lax.dot_general(rt_ref[...], prod,
                              (((1,), (0,)), ((), ())),
                              preferred_element_type=jnp.float32)
        out_ref[...] = jnp.concatenate(
            [msg,
             jnp.ones((1, eb), jnp.float32),
             jnp.zeros((16 - o - 1, eb), jnp.float32)], axis=0)

    return pl.pallas_call(
        body,
        grid=(grid,),
        in_specs=[
            pl.BlockSpec((d, eb), lambda i: (0, i)),
            pl.BlockSpec((xs_t.shape[0], eb), lambda i: (0, i)),
            pl.BlockSpec(w.shape, lambda i: (0, 0)),
            pl.BlockSpec(cvec.shape, lambda i: (0, 0)),
            pl.BlockSpec(rt.shape, lambda i: (0, 0)),
        ],
        out_specs=pl.BlockSpec((16, eb), lambda i: (0, i)),
        out_shape=jax.ShapeDtypeStruct((16, e), jnp.float32),
    )(pseudo_t, xs_t, w, cvec, rt)


def _tc_combine(agg2, x, root, bias):
    n, i_f = x.shape
    o = root.shape[0]
    nb = 1000
    grid = n // nb
    bias2 = bias[None, :]

    def body(a_ref, x_ref, root_ref, b_ref, out_ref):
        a = a_ref[...]
        s = a[0] + a[1]
        msg = s[:, 0:o]
        deg = s[:, o:o + 1]
        dense = lax.dot_general(x_ref[...], root_ref[...],
                                (((1,), (1,)), ((), ())),
                                preferred_element_type=jnp.float32)
        out_ref[...] = msg / jnp.maximum(deg, 1.0) + dense + b_ref[...]

    return pl.pallas_call(
        body,
        grid=(grid,),
        in_specs=[
            pl.BlockSpec((2, nb, 16), lambda i: (0, i, 0)),
            pl.BlockSpec((nb, i_f), lambda i: (i, 0)),
            pl.BlockSpec(root.shape, lambda i: (0, 0)),
            pl.BlockSpec((1, o), lambda i: (0, 0)),
        ],
        out_specs=pl.BlockSpec((nb, o), lambda i: (i, 0)),
        out_shape=jax.ShapeDtypeStruct((n, o), jnp.float32),
    )(agg2, x, root, bias2)


def kernel(edge_index, pseudo, x, mean, covariance, root, bias):
    e = edge_index.shape[1]
    n, i_f = x.shape
    o, _, k, d = mean.shape
    ew = e // NW
    nch = ew // CH
    src2 = edge_index[0].reshape(NW, ew)
    dst3 = edge_index[1].reshape(NW, nch, CH)

    mu = jnp.transpose(mean, (2, 0, 1, 3)).reshape(k * o * i_f, d)
    iv = 1.0 / (2.0 * jnp.transpose(covariance, (2, 0, 1, 3)
                                    ).reshape(k * o * i_f, d) ** 2 + 1e-8)
    w = jnp.concatenate([iv, -2.0 * mu * iv], axis=1)
    cvec = jnp.sum(mu * mu * iv, axis=1)[:, None]
    rt = jnp.repeat(jnp.eye(o, dtype=jnp.float32), i_f, axis=1) / k

    xs_t = _sc_gather(src2, x.T)
    msg16_t = _tc_dense(pseudo.T, xs_t, w, cvec, rt)
    agg2 = _sc_scatter(dst3, msg16_t, n)
    return _tc_combine(agg2, x, root, bias)


# speedup vs baseline: 1.5546x; 1.1443x over previous
import functools

import jax
import jax.numpy as jnp
from jax import lax
from jax.experimental import pallas as pl
from jax.experimental.pallas import tpu as pltpu
from jax.experimental.pallas import tpu_sc as plsc

NC = 2
NS = 16
NW = NC * NS
CH = 125


def _sc_gather(src2, xt):
    nw, ew = src2.shape
    f, n = xt.shape
    ewp = ((ew + 15) // 16) * 16
    ngr = ewp // 16
    mesh = plsc.VectorSubcoreMesh(core_axis_name="c", subcore_axis_name="s")

    @functools.partial(
        pl.kernel,
        out_type=jax.ShapeDtypeStruct((f, nw * ew), jnp.float32),
        mesh=mesh,
        compiler_params=pltpu.CompilerParams(use_tc_tiling_on_sc=False, needs_layout_passes=False),
        scratch_types=[
            pltpu.VMEM((f, n), jnp.float32),
            pltpu.VMEM((ewp,), jnp.int32),
            pltpu.VMEM((f, ewp), jnp.float32),
        ],
    )
    def gather_kernel(src_hbm, xt_hbm, xst_hbm, xt_v, idx_v, xst_v):
        wid = lax.axis_index("s") * NC + lax.axis_index("c")
        pltpu.sync_copy(xt_hbm, xt_v)
        pltpu.sync_copy(src_hbm.at[wid], idx_v.at[pl.ds(0, ew)])
        lanes = lax.iota(jnp.int32, 16)
        tail = idx_v[pl.ds(ewp - 16, 16)]
        idx_v[pl.ds(ewp - 16, 16)] = jnp.where(lanes < 16 - (ewp - ew),
                                               tail, 0)

        @plsc.parallel_loop(0, ngr, unroll=4)
        def _(q):
            base = q * 16
            idx16 = idx_v[pl.ds(base, 16)]
            for ff in range(f):
                row = jnp.full((16,), ff, jnp.int32)
                xst_v[ff, pl.ds(base, 16)] = plsc.load_gather(
                    xt_v, [row, idx16])
        pltpu.sync_copy(xst_v.at[:, pl.ds(0, ew)],
                        xst_hbm.at[:, pl.ds(wid * ew, ew)])

    return gather_kernel(src2, xt)


def _sc_scatter(dst3, msgt, n):
    nw, nch, ch = dst3.shape
    ew = nch * ch
    hch = 8
    nh = nch // hch
    hew = hch * ch
    unr = 8
    rpt = n // NS
    span0 = ((n // NS + 7) // 8) * 8
    last = n - span0 * (NS - 1)
    rptp = ((span0 + 15) // 16) * 16
    mesh = plsc.VectorSubcoreMesh(core_axis_name="c", subcore_axis_name="s")

    @functools.partial(
        pl.kernel,
        out_type=jax.ShapeDtypeStruct((NC, 16, n), jnp.float32),
        mesh=mesh,
        compiler_params=pltpu.CompilerParams(use_tc_tiling_on_sc=False, needs_layout_passes=False),
        scratch_types=[
            pltpu.VMEM((nch, ch), jnp.int32),
            pltpu.VMEM((16, hew), jnp.float32),
            pltpu.VMEM((hch, ch, 16), jnp.float32),
            pltpu.VMEM((rptp, 16), jnp.float32),
            pltpu.VMEM((16, rptp + 1), jnp.float32),
            pltpu.VMEM_SHARED((n, 16), jnp.float32),
            pltpu.SemaphoreType.DMA,
        ],
    )
    def scatter_kernel(dst_hbm, msgt_hbm, out_hbm, idx_v, msgt_v, msg_v,
                       zero_v, aggt_v, agg_sh, sem):
        cid = lax.axis_index("c")
        sid = lax.axis_index("s")
        wid = sid * NC + cid
        lanes = lax.iota(jnp.int32, 16)

        def zbody(i, carry):
            zero_v[i] = jnp.zeros((16,), jnp.float32)
            return carry

        lax.fori_loop(0, rptp, zbody, 0)
        pltpu.sync_copy(zero_v.at[pl.ds(0, rpt)],
                        agg_sh.at[pl.ds(sid * rpt, rpt)])
        pltpu.sync_copy(dst_hbm.at[wid], idx_v)
        plsc.subcore_barrier()

        for h in range(nh):
            pltpu.sync_copy(
                msgt_hbm.at[:, pl.ds(wid * ew + h * hew, hew)], msgt_v)

            @plsc.parallel_loop(0, hch * ch, unroll=unr)
            def _(e):
                col = jnp.full((16,), e, jnp.int32)
                v = plsc.load_gather(msgt_v, [lanes, col])
                msg_v[e // ch, e % ch] = v

            def sbody(j, carry):
                pltpu.async_copy(msg_v.at[j],
                                 agg_sh.at[idx_v.at[h * hch + j]],
                                 sem, add=True)
                return carry

            lax.fori_loop(0, hch, sbody, 0)

            def dbody(j, carry):
                pltpu.make_async_copy(msg_v.at[j],
                                      agg_sh.at[idx_v.at[h * hch + j]],
                                      sem).wait()
                return carry

            lax.fori_loop(0, hch, dbody, 0)

        plsc.subcore_barrier()

        off = sid * span0

        @pl.when(sid < NS - 1)
        def _():
            pltpu.sync_copy(agg_sh.at[pl.ds(off, span0)],
                            zero_v.at[pl.ds(0, span0)])

        @pl.when(sid == NS - 1)
        def _():
            pltpu.sync_copy(agg_sh.at[pl.ds(off, last)],
                            zero_v.at[pl.ds(0, last)])

        @plsc.parallel_loop(0, rptp, unroll=unr)
        def _(r):
            v = zero_v[r]
            plsc.store_scatter(aggt_v, [lanes, jnp.full((16,), r, jnp.int32)],
                               v)

        @pl.when(sid < NS - 1)
        def _():
            pltpu.sync_copy(aggt_v.at[:, pl.ds(0, span0)],
                            out_hbm.at[cid, :, pl.ds(off, span0)])

        @pl.when(sid == NS - 1)
        def _():
            pltpu.sync_copy(aggt_v.at[:, pl.ds(0, last)],
                            out_hbm.at[cid, :, pl.ds(off, last)])

    return scatter_kernel(dst3, msgt)


def _tc_dense(pseudo_t, xs_t, w, cvec, rt):
    d, e = pseudo_t.shape
    koi = w.shape[0]
    o, oi = rt.shape
    k = koi // oi
    eb = 32000
    grid = e // eb

    def body(p_ref, xs_ref, w_ref, c_ref, rt_ref, out_ref):
        p = p_ref[...]
        fmat = jnp.concatenate([p * p, p], axis=0)
        arg = lax.dot_general(w_ref[...], fmat,
                              (((1,), (0,)), ((), ())),
                              preferred_element_type=jnp.float32)
        g = jnp.exp(-(arg + c_ref[...]))
        gm = g[0:oi]
        for kk in range(1, k):
            gm = gm + g[kk * oi:(kk + 1) * oi]
        prod = gm * jnp.tile(xs_ref[...], (o, 1))
        msg = lax.dot_general(rt_ref[...], prod,
                              (((1,), (0,)), ((), ())),
                              preferred_element_type=jnp.float32)
        out_ref[...] = jnp.concatenate(
            [msg,
             jnp.ones((1, eb), jnp.float32),
             jnp.zeros((16 - o - 1, eb), jnp.float32)], axis=0)

    return pl.pallas_call(
        body,
        grid=(grid,),
        in_specs=[
            pl.BlockSpec((d, eb), lambda i: (0, i)),
            pl.BlockSpec((xs_t.shape[0], eb), lambda i: (0, i)),
            pl.BlockSpec(w.shape, lambda i: (0, 0)),
            pl.BlockSpec(cvec.shape, lambda i: (0, 0)),
            pl.BlockSpec(rt.shape, lambda i: (0, 0)),
        ],
        out_specs=pl.BlockSpec((16, eb), lambda i: (0, i)),
        out_shape=jax.ShapeDtypeStruct((16, e), jnp.float32),
    )(pseudo_t, xs_t, w, cvec, rt)


def _tc_combine(agg2t, xt, root, bias):
    i_f, n = xt.shape
    o = root.shape[0]
    bias2 = bias[:, None]

    def body(a_ref, xt_ref, root_ref, b_ref, out_ref):
        a = a_ref[...]
        s = a[0] + a[1]
        msg = s[0:o]
        deg = s[o:o + 1]
        dense = lax.dot_general(root_ref[...], xt_ref[...],
                                (((1,), (0,)), ((), ())),
                                preferred_element_type=jnp.float32)
        out_ref[...] = msg / jnp.maximum(deg, 1.0) + dense + b_ref[...]

    return pl.pallas_call(
        body,
        out_shape=jax.ShapeDtypeStruct((o, n), jnp.float32),
    )(agg2t, xt, root, bias2)


def kernel(edge_index, pseudo, x, mean, covariance, root, bias):
    e = edge_index.shape[1]
    n, i_f = x.shape
    o, _, k, d = mean.shape
    ew = e // NW
    nch = ew // CH
    src2 = edge_index[0].reshape(NW, ew)
    dst3 = edge_index[1].reshape(NW, nch, CH)

    mu = jnp.transpose(mean, (2, 0, 1, 3)).reshape(k * o * i_f, d)
    iv = 1.0 / (2.0 * jnp.transpose(covariance, (2, 0, 1, 3)
                                    ).reshape(k * o * i_f, d) ** 2 + 1e-8)
    w = jnp.concatenate([iv, -2.0 * mu * iv], axis=1)
    cvec = jnp.sum(mu * mu * iv, axis=1)[:, None]
    rt = jnp.repeat(jnp.eye(o, dtype=jnp.float32), i_f, axis=1) / k

    xt = x.T
    xs_t = _sc_gather(src2, xt)
    msg16_t = _tc_dense(pseudo.T, xs_t, w, cvec, rt)
    agg2t = _sc_scatter(dst3, msg16_t, n)
    return _tc_combine(agg2t, xt, root, bias).T
